# baseline (speedup 1.0000x reference)
import functools

import jax
import jax.numpy as jnp
from jax.experimental import pallas as pl
from jax.experimental.pallas import tpu as pltpu

_B, _S, _N = 32, 96, 2048
_E, _P = 8, 96
_BBLK = 4


def _moe_body(x_ref, wgblk_ref, waug_ref, out_ref, gate_ref):
    bb = pl.program_id(0)

    X_all = x_ref[...].reshape(_BBLK * _S, _N)
    G_all = jnp.dot(wgblk_ref[...], X_all, preferred_element_type=jnp.float32)

    for j in range(_BBLK):
        X = x_ref[j]
        G = G_all[j * _E:(j + 1) * _E]

        G = G - jnp.max(G, axis=0, keepdims=True)
        Pex = jnp.exp(G)
        Pr = Pex / jnp.sum(Pex, axis=0, keepdims=True)

        @pl.when((bb == 0) & (j == 0))
        def _():
            gate_ref[...] = Pr * (1.0 / _B)

        @pl.when((bb != 0) | (j != 0))
        def _():
            gate_ref[...] += Pr * (1.0 / _B)

        iota = jax.lax.broadcasted_iota(jnp.int32, (_E, Pr.shape[1]), 0)
        m1 = jnp.max(Pr, axis=0, keepdims=True)
        i1 = jnp.min(jnp.where(Pr == m1, iota, _E), axis=0, keepdims=True)
        Pm = jnp.where(iota == i1, -jnp.inf, Pr)
        m2 = jnp.max(Pm, axis=0, keepdims=True)
        i2 = jnp.min(jnp.where(Pm == m2, iota, _E), axis=0, keepdims=True)
        w = jnp.where((iota == i1) | (iota == i2), Pr, 0.0)

        X_big = (w[:, None, :] * X[None, :, :]).reshape(_E * _S, X.shape[1])
        X_aug = jnp.concatenate([X_big, w], axis=0)
        out_ref[j] = jnp.dot(waug_ref[...], X_aug,
                             preferred_element_type=jnp.float32)


@functools.partial(jax.jit, static_argnames=())
def kernel(x, Wg, We, be):
    wrow = We.transpose(1, 0, 2).reshape(_P, _E * _S)
    waug = jnp.concatenate([wrow, be.T], axis=1)
    wgblk = jnp.zeros((_BBLK * _E, _BBLK * _S), dtype=Wg.dtype)
    for j in range(_BBLK):
        wgblk = wgblk.at[j * _E:(j + 1) * _E, j * _S:(j + 1) * _S].set(Wg)
    out, gate_t = pl.pallas_call(
        _moe_body,
        grid=(_B // _BBLK,),
        in_specs=[
            pl.BlockSpec((_BBLK, _S, _N), lambda b: (b, 0, 0)),
            pl.BlockSpec((_BBLK * _E, _BBLK * _S), lambda b: (0, 0)),
            pl.BlockSpec((_P, _E * _S + _E), lambda b: (0, 0)),
        ],
        out_specs=[
            pl.BlockSpec((_BBLK, _P, _N), lambda b: (b, 0, 0)),
            pl.BlockSpec((_E, _N), lambda b: (0, 0)),
        ],
        out_shape=[
            jax.ShapeDtypeStruct((_B, _P, _N), jnp.float32),
            jax.ShapeDtypeStruct((_E, _N), jnp.float32),
        ],
        compiler_params=pltpu.CompilerParams(
            dimension_semantics=("arbitrary",),
        ),
    )(x, wgblk, waug)
    return out, gate_t.T


# speedup vs baseline: 1.0287x; 1.0287x over previous
import functools

import jax
import jax.numpy as jnp
from jax.experimental import pallas as pl
from jax.experimental.pallas import tpu as pltpu

_B, _S, _N = 32, 96, 2048
_E, _P = 8, 96
_BBLK = 4


def _moe_body(x_ref, wgblk_ref, wrow_ref, bet_ref, out_ref, gate_ref):
    bb = pl.program_id(0)

    X_all = x_ref[...].reshape(_BBLK * _S, _N)
    G_all = jnp.dot(wgblk_ref[...], X_all, preferred_element_type=jnp.float32)

    for j in range(_BBLK):
        X = x_ref[j]
        G = G_all[j * _E:(j + 1) * _E]

        G = G - jnp.max(G, axis=0, keepdims=True)
        Pex = jnp.exp(G)
        Pr = Pex / jnp.sum(Pex, axis=0, keepdims=True)

        @pl.when((bb == 0) & (j == 0))
        def _():
            gate_ref[...] = Pr * (1.0 / _B)

        @pl.when((bb != 0) | (j != 0))
        def _():
            gate_ref[...] += Pr * (1.0 / _B)

        iota = jax.lax.broadcasted_iota(jnp.int32, (_E, Pr.shape[1]), 0)
        m1 = jnp.max(Pr, axis=0, keepdims=True)
        i1 = jnp.min(jnp.where(Pr == m1, iota, _E), axis=0, keepdims=True)
        Pm = jnp.where(iota == i1, -jnp.inf, Pr)
        m2 = jnp.max(Pm, axis=0, keepdims=True)
        i2 = jnp.min(jnp.where(Pm == m2, iota, _E), axis=0, keepdims=True)
        w = jnp.where((iota == i1) | (iota == i2), Pr, 0.0)

        X_big = (w[:, None, :] * X[None, :, :]).reshape(_E * _S, X.shape[1])
        acc = jnp.dot(wrow_ref[...], X_big, preferred_element_type=jnp.float32)
        acc = acc + jnp.dot(bet_ref[...], w, preferred_element_type=jnp.float32)
        out_ref[j] = acc


@functools.partial(jax.jit, static_argnames=())
def kernel(x, Wg, We, be):
    wrow = We.transpose(1, 0, 2).reshape(_P, _E * _S)
    bet = be.T
    wgblk = jnp.zeros((_BBLK * _E, _BBLK * _S), dtype=Wg.dtype)
    for j in range(_BBLK):
        wgblk = wgblk.at[j * _E:(j + 1) * _E, j * _S:(j + 1) * _S].set(Wg)
    out, gate_t = pl.pallas_call(
        _moe_body,
        grid=(_B // _BBLK,),
        in_specs=[
            pl.BlockSpec((_BBLK, _S, _N), lambda b: (b, 0, 0)),
            pl.BlockSpec((_BBLK * _E, _BBLK * _S), lambda b: (0, 0)),
            pl.BlockSpec((_P, _E * _S), lambda b: (0, 0)),
            pl.BlockSpec((_P, _E), lambda b: (0, 0)),
        ],
        out_specs=[
            pl.BlockSpec((_BBLK, _P, _N), lambda b: (b, 0, 0)),
            pl.BlockSpec((_E, _N), lambda b: (0, 0)),
        ],
        out_shape=[
            jax.ShapeDtypeStruct((_B, _P, _N), jnp.float32),
            jax.ShapeDtypeStruct((_E, _N), jnp.float32),
        ],
        compiler_params=pltpu.CompilerParams(
            dimension_semantics=("arbitrary",),
        ),
    )(x, wgblk, wrow, bet)
    return out, gate_t.T


# speedup vs baseline: 1.1477x; 1.1157x over previous
import functools

import jax
import jax.numpy as jnp
from jax.experimental import pallas as pl
from jax.experimental.pallas import tpu as pltpu

_B, _S, _N = 32, 96, 2048
_E, _P = 8, 96
_BBLK = 4


def _moe_body(x_ref, wg_ref, wrow_ref, bet_ref, out_ref, gate_ref):
    bb = pl.program_id(0)

    for j in range(_BBLK):
        X = x_ref[j]

        G = jnp.dot(wg_ref[...], X, preferred_element_type=jnp.float32)
        G = G - jnp.max(G, axis=0, keepdims=True)
        Pex = jnp.exp(G)
        Pr = Pex / jnp.sum(Pex, axis=0, keepdims=True)

        @pl.when((bb == 0) & (j == 0))
        def _():
            gate_ref[...] = Pr * (1.0 / _B)

        @pl.when((bb != 0) | (j != 0))
        def _():
            gate_ref[...] += Pr * (1.0 / _B)

        iota = jax.lax.broadcasted_iota(jnp.int32, (_E, Pr.shape[1]), 0)
        m1 = jnp.max(Pr, axis=0, keepdims=True)
        i1 = jnp.min(jnp.where(Pr == m1, iota, _E), axis=0, keepdims=True)
        Pm = jnp.where(iota == i1, -jnp.inf, Pr)
        m2 = jnp.max(Pm, axis=0, keepdims=True)
        i2 = jnp.min(jnp.where(Pm == m2, iota, _E), axis=0, keepdims=True)
        w = jnp.where((iota == i1) | (iota == i2), Pr, 0.0)

        X_big = (w[:, None, :] * X[None, :, :]).reshape(_E * _S, X.shape[1])
        acc = jnp.dot(wrow_ref[...], X_big, preferred_element_type=jnp.float32)
        acc = acc + jnp.dot(bet_ref[...], w, preferred_element_type=jnp.float32)
        out_ref[j] = acc


@functools.partial(jax.jit, static_argnames=())
def kernel(x, Wg, We, be):
    wrow = We.transpose(1, 0, 2).reshape(_P, _E * _S)
    bet = be.T
    out, gate_t = pl.pallas_call(
        _moe_body,
        grid=(_B // _BBLK,),
        in_specs=[
            pl.BlockSpec((_BBLK, _S, _N), lambda b: (b, 0, 0)),
            pl.BlockSpec((_E, _S), lambda b: (0, 0)),
            pl.BlockSpec((_P, _E * _S), lambda b: (0, 0)),
            pl.BlockSpec((_P, _E), lambda b: (0, 0)),
        ],
        out_specs=[
            pl.BlockSpec((_BBLK, _P, _N), lambda b: (b, 0, 0)),
            pl.BlockSpec((_E, _N), lambda b: (0, 0)),
        ],
        out_shape=[
            jax.ShapeDtypeStruct((_B, _P, _N), jnp.float32),
            jax.ShapeDtypeStruct((_E, _N), jnp.float32),
        ],
        compiler_params=pltpu.CompilerParams(
            dimension_semantics=("arbitrary",),
        ),
    )(x, Wg, wrow, bet)
    return out, gate_t.T


# speedup vs baseline: 1.2332x; 1.0745x over previous
import functools

import jax
import jax.numpy as jnp
from jax.experimental import pallas as pl
from jax.experimental.pallas import tpu as pltpu

_B, _S, _N = 32, 96, 2048
_E, _P = 8, 96
_BBLK = 4


def _moe_body(x_ref, wg_ref, wrow_ref, bet_ref, out_ref, gate_ref):
    bb = pl.program_id(0)

    for j in range(_BBLK):
        X = x_ref[j]

        G = jnp.dot(wg_ref[...], X, preferred_element_type=jnp.float32)
        G = G - jnp.max(G, axis=0, keepdims=True)
        Pex = jnp.exp(G)
        Pr = Pex / jnp.sum(Pex, axis=0, keepdims=True)

        @pl.when((bb == 0) & (j == 0))
        def _():
            gate_ref[...] = Pr * (1.0 / _B)

        @pl.when((bb != 0) | (j != 0))
        def _():
            gate_ref[...] += Pr * (1.0 / _B)

        iota = jax.lax.broadcasted_iota(jnp.int32, (_E, Pr.shape[1]), 0)
        m1 = jnp.max(Pr, axis=0, keepdims=True)
        i1 = jnp.min(jnp.where(Pr == m1, iota, _E), axis=0, keepdims=True)
        Pm = jnp.where(iota == i1, -jnp.inf, Pr)
        m2 = jnp.max(Pm, axis=0, keepdims=True)
        i2 = jnp.min(jnp.where(Pm == m2, iota, _E), axis=0, keepdims=True)
        w = jnp.where((iota == i1) | (iota == i2), Pr, 0.0)

        w_bf = w.astype(jnp.bfloat16)
        X_bf = X.astype(jnp.bfloat16)
        X_big = (w_bf[:, None, :] * X_bf[None, :, :]).reshape(_E * _S, X.shape[1])
        acc = jnp.dot(wrow_ref[...], X_big, preferred_element_type=jnp.float32)
        acc = acc + jnp.dot(bet_ref[...], w, preferred_element_type=jnp.float32)
        out_ref[j] = acc


@functools.partial(jax.jit, static_argnames=())
def kernel(x, Wg, We, be):
    wrow = We.transpose(1, 0, 2).reshape(_P, _E * _S).astype(jnp.bfloat16)
    bet = be.T
    out, gate_t = pl.pallas_call(
        _moe_body,
        grid=(_B // _BBLK,),
        in_specs=[
            pl.BlockSpec((_BBLK, _S, _N), lambda b: (b, 0, 0)),
            pl.BlockSpec((_E, _S), lambda b: (0, 0)),
            pl.BlockSpec((_P, _E * _S), lambda b: (0, 0)),
            pl.BlockSpec((_P, _E), lambda b: (0, 0)),
        ],
        out_specs=[
            pl.BlockSpec((_BBLK, _P, _N), lambda b: (b, 0, 0)),
            pl.BlockSpec((_E, _N), lambda b: (0, 0)),
        ],
        out_shape=[
            jax.ShapeDtypeStruct((_B, _P, _N), jnp.float32),
            jax.ShapeDtypeStruct((_E, _N), jnp.float32),
        ],
        compiler_params=pltpu.CompilerParams(
            dimension_semantics=("arbitrary",),
        ),
    )(x, Wg, wrow, bet)
    return out, gate_t.T


# speedup vs baseline: 1.2582x; 1.0203x over previous
import functools

import jax
import jax.numpy as jnp
from jax.experimental import pallas as pl
from jax.experimental.pallas import tpu as pltpu

_B, _S, _N = 32, 96, 2048
_E, _P = 8, 96
_BBLK = 4


def _moe_body(x_ref, wg_ref, wrow_ref, bet_ref, out_ref, gate_ref):
    bb = pl.program_id(0)

    for j in range(_BBLK):
        X = x_ref[j]

        G = jnp.dot(wg_ref[...], X, preferred_element_type=jnp.float32)
        G = G - jnp.max(G, axis=0, keepdims=True)
        Pex = jnp.exp(G)
        Pr = Pex * (1.0 / jnp.sum(Pex, axis=0, keepdims=True))

        @pl.when((bb == 0) & (j == 0))
        def _():
            gate_ref[...] = Pr * (1.0 / _B)

        @pl.when((bb != 0) | (j != 0))
        def _():
            gate_ref[...] += Pr * (1.0 / _B)

        iota = jax.lax.broadcasted_iota(jnp.int32, (_E, Pr.shape[1]), 0)
        m1 = jnp.max(Pr, axis=0, keepdims=True)
        i1 = jnp.min(jnp.where(Pr == m1, iota, _E), axis=0, keepdims=True)
        Pm = jnp.where(iota == i1, -jnp.inf, Pr)
        m2 = jnp.max(Pm, axis=0, keepdims=True)
        i2 = jnp.min(jnp.where(Pm == m2, iota, _E), axis=0, keepdims=True)
        w = jnp.where((iota == i1) | (iota == i2), Pr, 0.0)

        w_bf = w.astype(jnp.bfloat16)
        X_bf = X.astype(jnp.bfloat16)
        X_big = (w_bf[:, None, :] * X_bf[None, :, :]).reshape(_E * _S, X.shape[1])
        acc = jnp.dot(wrow_ref[...], X_big, preferred_element_type=jnp.float32)
        acc = acc + jnp.dot(bet_ref[...], w, preferred_element_type=jnp.float32)
        out_ref[j] = acc


@functools.partial(jax.jit, static_argnames=())
def kernel(x, Wg, We, be):
    wrow = We.transpose(1, 0, 2).reshape(_P, _E * _S).astype(jnp.bfloat16)
    bet = be.T
    out, gate_t = pl.pallas_call(
        _moe_body,
        grid=(_B // _BBLK,),
        in_specs=[
            pl.BlockSpec((_BBLK, _S, _N), lambda b: (b, 0, 0)),
            pl.BlockSpec((_E, _S), lambda b: (0, 0)),
            pl.BlockSpec((_P, _E * _S), lambda b: (0, 0)),
            pl.BlockSpec((_P, _E), lambda b: (0, 0)),
        ],
        out_specs=[
            pl.BlockSpec((_BBLK, _P, _N), lambda b: (b, 0, 0)),
            pl.BlockSpec((_E, _N), lambda b: (0, 0)),
        ],
        out_shape=[
            jax.ShapeDtypeStruct((_B, _P, _N), jnp.float32),
            jax.ShapeDtypeStruct((_E, _N), jnp.float32),
        ],
        compiler_params=pltpu.CompilerParams(
            dimension_semantics=("arbitrary",),
        ),
    )(x, Wg, wrow, bet)
    return out, gate_t.T
